# baseline (speedup 1.0000x reference)
import functools

import jax
import jax.numpy as jnp
from jax import lax
from jax.experimental import pallas as pl
from jax.experimental.pallas import tpu as pltpu
from jax.experimental.pallas import tpu_sc as plsc

_N_NODES = 100000
_N_EDGES = 1600000
_NW = 32
_C = 512
_S = 128
_G = _C // 16
_NCHUNKS = _N_EDGES // _C
_CPW = -(-_NCHUNKS // _NW)

_HALF_PI = 1.5707963267948966

_ATAN_C = (
    0.9999999937572801, -0.3333313797588716, 0.19993694079563193,
    -0.14211102809331508, 0.10667470050577645, -0.07556856634693432,
    0.04327764436731928, -0.01641277490593999, 0.00293264657519318,
)


def _f32(x):
    return jnp.float32(x)


def _recip(x):
    i = plsc.bitcast(x, jnp.int32)
    i = jnp.int32(0x7EF311C3) - i
    r = plsc.bitcast(i, jnp.float32)
    for _ in range(3):
        r = r * (_f32(2.0) - x * r)
    return r


def _rsqrt(a):
    i = plsc.bitcast(a, jnp.int32)
    i = jnp.int32(0x5F3759DF) - (i >> 1)
    y = plsc.bitcast(i, jnp.float32)
    for _ in range(3):
        y = y * (_f32(1.5) - _f32(0.5) * a * y * y)
    return y


def _atan01(z):
    u = z * z
    p = _f32(_ATAN_C[8])
    for k in range(7, -1, -1):
        p = p * u + _f32(_ATAN_C[k])
    return p * z


def _qmul(a, b):
    ax, ay, az, aw = a
    bx, by, bz, bw = b
    return (aw * bx + ax * bw + ay * bz - az * by,
            aw * by - ax * bz + ay * bw + az * bx,
            aw * bz + ax * by - ay * bx + az * bw,
            aw * bw - ax * bx - ay * by - az * bz)


def _qrot(q, v):
    qx, qy, qz, qw = q
    vx, vy, vz = v
    ux = qy * vz - qz * vy
    uy = qz * vx - qx * vz
    uz = qx * vy - qy * vx
    wx = qy * uz - qz * uy
    wy = qz * ux - qx * uz
    wz = qx * uy - qy * ux
    return (vx + _f32(2.0) * (qw * ux + wx),
            vy + _f32(2.0) * (qw * uy + wy),
            vz + _f32(2.0) * (qw * uz + wz))


def _edge_math(tp, qp, t1, q1, t2, q2):
    qpc = (-qp[0], -qp[1], -qp[2], qp[3])
    q1c = (-q1[0], -q1[1], -q1[2], q1[3])
    u = (t2[0] - t1[0], t2[1] - t1[1], t2[2] - t1[2])
    v1 = _qrot(q1c, u)
    wv = (v1[0] - tp[0], v1[1] - tp[1], v1[2] - tp[2])
    tT = _qrot(qpc, wv)
    qT = _qmul(_qmul(qpc, q1c), q2)

    x, y, z, w = qT
    sgn = jnp.where(w < _f32(0.0), _f32(-1.0), _f32(1.0))
    x = x * sgn
    y = y * sgn
    z = z * sgn
    w = w * sgn
    n2 = x * x + y * y + z * z
    a = n2 + _f32(1e-24)
    ry = _rsqrt(a)
    n = a * ry
    mn = jnp.minimum(n, w)
    mx = jnp.maximum(n, w)
    t = _atan01(mn * _recip(mx))
    half = jnp.where(n > w, _f32(_HALF_PI) - t, t)
    angle = _f32(2.0) * half
    small = n < _f32(1e-7)
    factor = jnp.where(small, _f32(2.0), angle * ry)
    px = x * factor
    py = y * factor
    pz = z * factor
    theta2 = px * px + py * py + pz * pz
    small2 = angle < _f32(1e-6)
    th = jnp.where(small2, _f32(1.0), angle)
    rth = _recip(th)
    coef = jnp.where(small2, _f32(1.0 / 12.0),
                     rth * rth - _f32(0.5) * w * ry * rth)
    tx, ty, tz = tT
    pt = px * tx + py * ty + pz * tz
    aa = _f32(1.0) - coef * theta2
    cx = py * tz - pz * ty
    cy = pz * tx - px * tz
    cz = px * ty - py * tx
    cp = coef * pt
    r0 = tx * aa - _f32(0.5) * cx + cp * px
    r1 = ty * aa - _f32(0.5) * cy + cp * py
    r2 = tz * aa - _f32(0.5) * cz + cp * pz
    return (r0, r1, r2, px, py, pz)


def _sc_body(edges_hbm, poses_hbm, infos_hbm, nodes_hbm, out_hbm,
             *scratch):
    set_a = scratch[:8]
    set_b = scratch[8:]
    cid = lax.axis_index("c")
    sid = lax.axis_index("s")
    wid = sid * 2 + cid
    iota = lax.iota(jnp.int32, 16)
    cols = [jnp.full((16,), c, jnp.int32) for c in range(7)]

    def fire(st, c):
        idx1_v, idx2_v, pos_v, inf_v, n1_v, n2_v, _, sem = st
        base = c * _C
        blk = c * (_C // _S)
        a = pltpu.async_copy(edges_hbm.at[0, pl.ds(blk, _C // _S)], idx1_v, sem)
        b = pltpu.async_copy(edges_hbm.at[1, pl.ds(blk, _C // _S)], idx2_v, sem)
        a.wait()
        b.wait()
        pltpu.async_copy(poses_hbm.at[:, pl.ds(base, _C)], pos_v, sem)
        pltpu.async_copy(infos_hbm.at[:, pl.ds(base, _C)], inf_v, sem)
        for k in range(_C // _S):
            pltpu.async_copy(nodes_hbm.at[idx1_v.at[k]], n1_v.at[pl.ds(k * _S, _S)], sem)
            pltpu.async_copy(nodes_hbm.at[idx2_v.at[k]], n2_v.at[pl.ds(k * _S, _S)], sem)

    def drain(st, c):
        idx1_v, idx2_v, pos_v, inf_v, n1_v, n2_v, _, sem = st
        base = c * _C
        pltpu.make_async_copy(poses_hbm.at[:, pl.ds(base, _C)], pos_v, sem).wait()
        pltpu.make_async_copy(infos_hbm.at[:, pl.ds(base, _C)], inf_v, sem).wait()
        for k in range(_C // _S):
            pltpu.make_async_copy(
                nodes_hbm.at[idx1_v.at[k]], n1_v.at[pl.ds(k * _S, _S)], sem).wait()
            pltpu.make_async_copy(
                nodes_hbm.at[idx2_v.at[k]], n2_v.at[pl.ds(k * _S, _S)], sem).wait()

    def compute(st, c):
        _, _, pos_v, inf_v, n1_v, n2_v, out_v, _ = st
        base = c * _C

        def group(g, _):
            e0 = g * 16
            rows = iota + e0
            tp = [pos_v[c2, pl.ds(e0, 16)] for c2 in range(3)]
            qp = [pos_v[c2, pl.ds(e0, 16)] for c2 in range(3, 7)]
            nn1 = [plsc.load_gather(n1_v, [rows, cols[c2]]) for c2 in range(7)]
            nn2 = [plsc.load_gather(n2_v, [rows, cols[c2]]) for c2 in range(7)]
            r6 = _edge_math(tp, qp, nn1[:3], nn1[3:], nn2[:3], nn2[3:])
            for oi in range(6):
                s = inf_v[oi * 6, pl.ds(e0, 16)] * r6[0]
                for j in range(1, 6):
                    s = s + inf_v[oi * 6 + j, pl.ds(e0, 16)] * r6[j]
                out_v[oi, pl.ds(e0, 16)] = s

        lax.fori_loop(0, _G, group, None)
        pltpu.sync_copy(out_v, out_hbm.at[:, pl.ds(base, _C)])

    fire(set_a, wid)

    def pair_body(j, _):
        c0 = wid + (2 * j) * _NW
        c1 = c0 + _NW
        c2 = c0 + 2 * _NW

        @pl.when(c1 < _NCHUNKS)
        def _():
            fire(set_b, c1)

        drain(set_a, c0)
        compute(set_a, c0)

        @pl.when(c2 < _NCHUNKS)
        def _():
            fire(set_a, c2)

        @pl.when(c1 < _NCHUNKS)
        def _():
            drain(set_b, c1)
            compute(set_b, c1)

    lax.fori_loop(0, _CPW // 2, pair_body, None)


_mesh = plsc.VectorSubcoreMesh(core_axis_name="c", subcore_axis_name="s")

_sc_call = functools.partial(
    pl.kernel,
    out_type=jax.ShapeDtypeStruct((6, _N_EDGES), jnp.float32),
    mesh=_mesh,
    scratch_types=[
        pltpu.VMEM((_C // _S, _S), jnp.int32),
        pltpu.VMEM((_C // _S, _S), jnp.int32),
        pltpu.VMEM((7, _C), jnp.float32),
        pltpu.VMEM((36, _C), jnp.float32),
        pltpu.VMEM((_C, 16), jnp.float32),
        pltpu.VMEM((_C, 16), jnp.float32),
        pltpu.VMEM((6, _C), jnp.float32),
        pltpu.SemaphoreType.DMA,
    ] * 2,
    compiler_params=pltpu.CompilerParams(
        needs_layout_passes=False, use_tc_tiling_on_sc=False),
)(_sc_body)


def kernel(edges, poses, infos, nodes):
    edges_t = edges.astype(jnp.int32).T.reshape(2, _N_EDGES // _S, _S)
    poses_t = poses.T
    infos_t = infos.transpose(1, 2, 0).reshape(36, _N_EDGES)
    nodes_p = jnp.pad(nodes, ((0, 0), (0, 9)))
    out = _sc_call(edges_t, poses_t, infos_t, nodes_p)
    return out.T


# speedup vs baseline: 4.3248x; 4.3248x over previous
import functools

import jax
import jax.numpy as jnp
from jax import lax
from jax.experimental import pallas as pl
from jax.experimental.pallas import tpu as pltpu
from jax.experimental.pallas import tpu_sc as plsc

_N_NODES = 100000
_N_EDGES = 1600000
_NW = 32
_C = 512
_S = 128
_G = _C // 16
_NB = _C // _S
_NCHUNKS = _N_EDGES // _C
_CPW = -(-_NCHUNKS // _NW)
_NBLK = _N_EDGES // _S

_HALF_PI = 1.5707963267948966

_ATAN_C = (
    0.9999999937572801, -0.3333313797588716, 0.19993694079563193,
    -0.14211102809331508, 0.10667470050577645, -0.07556856634693432,
    0.04327764436731928, -0.01641277490593999, 0.00293264657519318,
)


def _f32(x):
    return jnp.float32(x)


def _recip(x):
    i = plsc.bitcast(x, jnp.int32)
    i = jnp.int32(0x7EF311C3) - i
    r = plsc.bitcast(i, jnp.float32)
    for _ in range(3):
        r = r * (_f32(2.0) - x * r)
    return r


def _rsqrt(a):
    i = plsc.bitcast(a, jnp.int32)
    i = jnp.int32(0x5F3759DF) - (i >> 1)
    y = plsc.bitcast(i, jnp.float32)
    for _ in range(3):
        y = y * (_f32(1.5) - _f32(0.5) * a * y * y)
    return y


def _atan01(z):
    u = z * z
    p = _f32(_ATAN_C[8])
    for k in range(7, -1, -1):
        p = p * u + _f32(_ATAN_C[k])
    return p * z


def _qmul(a, b):
    ax, ay, az, aw = a
    bx, by, bz, bw = b
    return (aw * bx + ax * bw + ay * bz - az * by,
            aw * by - ax * bz + ay * bw + az * bx,
            aw * bz + ax * by - ay * bx + az * bw,
            aw * bw - ax * bx - ay * by - az * bz)


def _qrot(q, v):
    qx, qy, qz, qw = q
    vx, vy, vz = v
    ux = qy * vz - qz * vy
    uy = qz * vx - qx * vz
    uz = qx * vy - qy * vx
    wx = qy * uz - qz * uy
    wy = qz * ux - qx * uz
    wz = qx * uy - qy * ux
    return (vx + _f32(2.0) * (qw * ux + wx),
            vy + _f32(2.0) * (qw * uy + wy),
            vz + _f32(2.0) * (qw * uz + wz))


def _edge_math(tp, qp, t1, q1, t2, q2):
    qpc = (-qp[0], -qp[1], -qp[2], qp[3])
    q1c = (-q1[0], -q1[1], -q1[2], q1[3])
    u = (t2[0] - t1[0], t2[1] - t1[1], t2[2] - t1[2])
    v1 = _qrot(q1c, u)
    wv = (v1[0] - tp[0], v1[1] - tp[1], v1[2] - tp[2])
    tT = _qrot(qpc, wv)
    qT = _qmul(_qmul(qpc, q1c), q2)

    x, y, z, w = qT
    sgn = jnp.where(w < _f32(0.0), _f32(-1.0), _f32(1.0))
    x = x * sgn
    y = y * sgn
    z = z * sgn
    w = w * sgn
    n2 = x * x + y * y + z * z
    a = n2 + _f32(1e-24)
    ry = _rsqrt(a)
    n = a * ry
    mn = jnp.minimum(n, w)
    mx = jnp.maximum(n, w)
    t = _atan01(mn * _recip(mx))
    half = jnp.where(n > w, _f32(_HALF_PI) - t, t)
    angle = _f32(2.0) * half
    small = n < _f32(1e-7)
    factor = jnp.where(small, _f32(2.0), angle * ry)
    px = x * factor
    py = y * factor
    pz = z * factor
    theta2 = px * px + py * py + pz * pz
    small2 = angle < _f32(1e-6)
    th = jnp.where(small2, _f32(1.0), angle)
    rth = _recip(th)
    coef = jnp.where(small2, _f32(1.0 / 12.0),
                     rth * rth - _f32(0.5) * w * ry * rth)
    tx, ty, tz = tT
    pt = px * tx + py * ty + pz * tz
    aa = _f32(1.0) - coef * theta2
    cx = py * tz - pz * ty
    cy = pz * tx - px * tz
    cz = px * ty - py * tx
    cp = coef * pt
    r0 = tx * aa - _f32(0.5) * cx + cp * px
    r1 = ty * aa - _f32(0.5) * cy + cp * py
    r2 = tz * aa - _f32(0.5) * cz + cp * pz
    return (r0, r1, r2, px, py, pz)


def _sc_body(edges_hbm, poses_hbm, nodes_hbm, out_hbm, *scratch):
    set_a = scratch[:7]
    set_b = scratch[7:]
    cid = lax.axis_index("c")
    sid = lax.axis_index("s")
    wid = sid * 2 + cid
    iota = lax.iota(jnp.int32, 16)
    cols = [jnp.full((16,), c, jnp.int32) for c in range(7)]

    def fire(st, c):
        idx1_v, idx2_v, pos_v, n1_v, n2_v, _, sem = st
        cps = []
        for k in range(_NB):
            cps.append(pltpu.async_copy(
                edges_hbm.at[pl.ds(c * _C * 2 + k * 2 * _S, _S)],
                idx1_v.at[k], sem))
            cps.append(pltpu.async_copy(
                edges_hbm.at[pl.ds(c * _C * 2 + (k * 2 + 1) * _S, _S)],
                idx2_v.at[k], sem))
        for cp in cps:
            cp.wait()
        pltpu.async_copy(poses_hbm.at[pl.ds(c * _NB, _NB)], pos_v, sem)
        for k in range(_NB):
            pltpu.async_copy(nodes_hbm.at[idx1_v.at[k]],
                             n1_v.at[pl.ds(k * _S, _S)], sem)
            pltpu.async_copy(nodes_hbm.at[idx2_v.at[k]],
                             n2_v.at[pl.ds(k * _S, _S)], sem)

    def drain(st, c):
        idx1_v, idx2_v, pos_v, n1_v, n2_v, _, sem = st
        pltpu.make_async_copy(poses_hbm.at[pl.ds(c * _NB, _NB)], pos_v,
                              sem).wait()
        for k in range(_NB):
            pltpu.make_async_copy(nodes_hbm.at[idx1_v.at[k]],
                                  n1_v.at[pl.ds(k * _S, _S)], sem).wait()
            pltpu.make_async_copy(nodes_hbm.at[idx2_v.at[k]],
                                  n2_v.at[pl.ds(k * _S, _S)], sem).wait()

    def compute(st, c):
        _, _, pos_v, n1_v, n2_v, out_v, _ = st

        def group(g, _):
            b = g // 8
            l0 = (g % 8) * 16
            rows = iota + g * 16
            tp = [pos_v[b, pl.ds(c2 * 128 + l0, 16)] for c2 in range(3)]
            qp = [pos_v[b, pl.ds(c2 * 128 + l0, 16)] for c2 in range(3, 7)]
            nn1 = [plsc.load_gather(n1_v, [rows, cols[c2]]) for c2 in range(7)]
            nn2 = [plsc.load_gather(n2_v, [rows, cols[c2]]) for c2 in range(7)]
            r6 = _edge_math(tp, qp, nn1[:3], nn1[3:], nn2[:3], nn2[3:])
            for oi in range(6):
                out_v[b, pl.ds(oi * 128 + l0, 16)] = r6[oi]

        lax.fori_loop(0, _G, group, None)
        pltpu.sync_copy(out_v, out_hbm.at[pl.ds(c * _NB, _NB)])

    fire(set_a, wid)

    def pair_body(j, _):
        c0 = wid + (2 * j) * _NW
        c1 = c0 + _NW
        c2 = c0 + 2 * _NW

        @pl.when(c1 < _NCHUNKS)
        def _():
            fire(set_b, c1)

        drain(set_a, c0)
        compute(set_a, c0)

        @pl.when(c2 < _NCHUNKS)
        def _():
            fire(set_a, c2)

        @pl.when(c1 < _NCHUNKS)
        def _():
            drain(set_b, c1)
            compute(set_b, c1)

    lax.fori_loop(0, _CPW // 2, pair_body, None)


_mesh = plsc.VectorSubcoreMesh(core_axis_name="c", subcore_axis_name="s")

_sc_call = functools.partial(
    pl.kernel,
    out_type=jax.ShapeDtypeStruct((_NBLK, 1024), jnp.float32),
    mesh=_mesh,
    scratch_types=[
        pltpu.VMEM((_NB, _S), jnp.int32),
        pltpu.VMEM((_NB, _S), jnp.int32),
        pltpu.VMEM((_NB, 1024), jnp.float32),
        pltpu.VMEM((_C, 16), jnp.float32),
        pltpu.VMEM((_C, 16), jnp.float32),
        pltpu.VMEM((_NB, 1024), jnp.float32),
        pltpu.SemaphoreType.DMA,
    ] * 2,
    compiler_params=pltpu.CompilerParams(
        needs_layout_passes=False, use_tc_tiling_on_sc=False),
)(_sc_body)


_TBLK = 1024


def _tc_matvec_body(inf_ref, r6_ref, out_ref):
    outs = []
    for i in range(6):
        s = inf_ref[i, 0, :].reshape(_TBLK // 128, 128) * r6_ref[:, pl.ds(0, 128)]
        for j in range(1, 6):
            s = s + (inf_ref[i, j, :].reshape(_TBLK // 128, 128)
                     * r6_ref[:, pl.ds(j * 128, 128)])
        outs.append(s.reshape(1, _TBLK))
    out_ref[...] = jnp.concatenate(outs, axis=0)


def _tc_matvec(infos_t, r6):
    return pl.pallas_call(
        _tc_matvec_body,
        out_shape=jax.ShapeDtypeStruct((6, _N_EDGES), jnp.float32),
        grid=(pl.cdiv(_N_EDGES, _TBLK),),
        in_specs=[
            pl.BlockSpec((6, 6, _TBLK), lambda b: (0, 0, b)),
            pl.BlockSpec((_TBLK // 128, 1024), lambda b: (b, 0)),
        ],
        out_specs=pl.BlockSpec((6, _TBLK), lambda b: (0, b)),
        compiler_params=pltpu.CompilerParams(
            dimension_semantics=("arbitrary",)),
    )(infos_t, r6)


def kernel(edges, poses, infos, nodes):
    edges_lin = (edges.astype(jnp.int32)
                 .reshape(_NBLK, _S, 2).transpose(0, 2, 1).reshape(-1))
    poses_til = (jnp.pad(poses, ((0, 0), (0, 1)))
                 .reshape(_NBLK, _S, 8).transpose(0, 2, 1).reshape(_NBLK, 1024))
    nodes_p = jnp.pad(nodes, ((0, 0), (0, 9)))
    infos_t = infos.transpose(1, 2, 0)
    r6 = _sc_call(edges_lin, poses_til, nodes_p)
    out = _tc_matvec(infos_t, r6)
    return out.T
